# Initial kernel scaffold; baseline (speedup 1.0000x reference)
import jax
import jax.numpy as jnp
from jax.experimental import pallas as pl


def kernel(dense_feature, sparse_feature, table, Ww, bw, W0, b0, W1, b1, W2, b2, W3, b3, W4, b4):
    raise NotImplementedError("write your pallas kernel here")


# speedup vs baseline: 15.2906x; 15.2906x over previous
import functools

import jax
import jax.numpy as jnp
from jax import lax
from jax.experimental import pallas as pl
from jax.experimental.pallas import tpu as pltpu
from jax.experimental.pallas import tpu_sc as plsc

B = 16384
V = 1000000
D = 16
NS = 26
DENSE = 13

NW = 32
IDX_COLS = 128
IDX_ROWS = (B * NS) // IDX_COLS
ROWS_PER_W = IDX_ROWS // NW
CHUNK = 8
N_CHUNKS = ROWS_PER_W // CHUNK


def _sc_gather(table, idx2d):
    mesh = plsc.VectorSubcoreMesh(core_axis_name="c", subcore_axis_name="s")

    @functools.partial(
        pl.kernel,
        mesh=mesh,
        out_type=jax.ShapeDtypeStruct((IDX_ROWS, IDX_COLS, D), jnp.float32),
        scratch_types=[
            pltpu.VMEM((CHUNK, IDX_COLS), jnp.int32),
            pltpu.VMEM((CHUNK, IDX_COLS, D), jnp.float32),
            pltpu.SemaphoreType.DMA,
        ],
        compiler_params=pltpu.CompilerParams(use_tc_tiling_on_sc=False),
    )
    def k(table_hbm, idx_hbm, out_hbm, idx_v, rows_v, sem):
        wid = lax.axis_index("s") * 2 + lax.axis_index("c")
        base = wid * ROWS_PER_W
        for i in range(N_CHUNKS):
            r0 = base + i * CHUNK
            pltpu.sync_copy(idx_hbm.at[pl.ds(r0, CHUNK)], idx_v)

            def fire(j, carry):
                pltpu.async_copy(table_hbm.at[idx_v.at[j]], rows_v.at[j], sem)
                return carry

            lax.fori_loop(0, CHUNK, fire, 0)
            pltpu.make_async_copy(out_hbm.at[pl.ds(r0, CHUNK)], rows_v, sem).wait()
            pltpu.sync_copy(rows_v, out_hbm.at[pl.ds(r0, CHUNK)])

    return k(table, idx2d)


def _mlp_body(emb_ref, den_ref, w0e, w0d, b0r, w1, b1r, w2, b2r, w3, b3r,
              w4, b4r, ww, bwr, out_ref):
    f32 = jnp.float32
    x = emb_ref[...]
    d = den_ref[...]
    h = jnp.dot(x, w0e[...], preferred_element_type=f32)
    h = h + jnp.dot(d, w0d[...], preferred_element_type=f32) + b0r[...]
    h = jnp.maximum(h, 0.0)
    h = jnp.maximum(jnp.dot(h, w1[...], preferred_element_type=f32) + b1r[...], 0.0)
    h = jnp.maximum(jnp.dot(h, w2[...], preferred_element_type=f32) + b2r[...], 0.0)
    h = jnp.maximum(jnp.dot(h, w3[...], preferred_element_type=f32) + b3r[...], 0.0)
    y = jnp.dot(h, w4[...], preferred_element_type=f32) + b4r[...]
    wide = jnp.dot(d, ww[...], preferred_element_type=f32) + bwr[...]
    out_ref[...] = y + wide


def _tc_mlp(emb, dense, w0eT, w0dT, b0, w1T, b1, w2T, b2, w3T, b3, w4T, b4,
            wwT, bw):
    BM = 2048
    grid = (B // BM,)

    def const(shape):
        return pl.BlockSpec(shape, lambda i: (0, 0))

    return pl.pallas_call(
        _mlp_body,
        grid=grid,
        in_specs=[
            pl.BlockSpec((BM, NS * D), lambda i: (i, 0)),
            pl.BlockSpec((BM, DENSE), lambda i: (i, 0)),
            const((NS * D, 512)),
            const((DENSE, 512)),
            const((1, 512)),
            const((512, 256)),
            const((1, 256)),
            const((256, 128)),
            const((1, 128)),
            const((128, 32)),
            const((1, 32)),
            const((32, 1)),
            const((1, 1)),
            const((DENSE, 1)),
            const((1, 1)),
        ],
        out_specs=pl.BlockSpec((BM, 1), lambda i: (i, 0)),
        out_shape=jax.ShapeDtypeStruct((B, 1), jnp.float32),
    )(emb, dense, w0eT, w0dT, b0, w1T, b1, w2T, b2, w3T, b3, w4T, b4, wwT, bw)


def kernel(dense_feature, sparse_feature, table, Ww, bw, W0, b0, W1, b1, W2,
           b2, W3, b3, W4, b4):
    idx2d = sparse_feature.T.reshape(IDX_ROWS, IDX_COLS)
    emb3d = _sc_gather(table, idx2d)
    emb = emb3d.reshape(B, NS * D)

    W0T = W0.T
    out = _tc_mlp(
        emb, dense_feature,
        W0T[: NS * D], W0T[NS * D:], b0.reshape(1, -1),
        W1.T, b1.reshape(1, -1),
        W2.T, b2.reshape(1, -1),
        W3.T, b3.reshape(1, -1),
        W4.T, b4.reshape(1, -1),
        Ww.T, bw.reshape(1, 1),
    )
    return out


# speedup vs baseline: 15.7365x; 1.0292x over previous
import functools

import jax
import jax.numpy as jnp
from jax import lax
from jax.experimental import pallas as pl
from jax.experimental.pallas import tpu as pltpu
from jax.experimental.pallas import tpu_sc as plsc

B = 16384
V = 1000000
D = 16
NS = 26
DENSE = 13

NW = 32
BPW = B // NW
IDX_COLS = 128
RPW = (BPW * NS) // IDX_COLS
CHUNK = 8
N_CHUNKS = RPW // CHUNK


def _sc_gather(table, sparse):
    mesh = plsc.VectorSubcoreMesh(core_axis_name="c", subcore_axis_name="s")
    out_rows = NW * RPW

    @functools.partial(
        pl.kernel,
        mesh=mesh,
        out_type=jax.ShapeDtypeStruct((out_rows, IDX_COLS, D), jnp.float32),
        scratch_types=[
            pltpu.VMEM((NS, BPW), jnp.int32),
            pltpu.VMEM((RPW, IDX_COLS), jnp.int32),
            pltpu.VMEM((2, CHUNK, IDX_COLS, D), jnp.float32),
            pltpu.SemaphoreType.DMA,
            pltpu.SemaphoreType.DMA,
        ],
        compiler_params=pltpu.CompilerParams(
            use_tc_tiling_on_sc=False, needs_layout_passes=False),
    )
    def k(table_hbm, idx_hbm, out_hbm, slab_v, idx_v, rows_v, sem_g, sem_o):
        wid = lax.axis_index("s") * 2 + lax.axis_index("c")
        base_b = wid * BPW
        base_r = wid * RPW

        pltpu.sync_copy(idx_hbm.at[:, pl.ds(base_b, BPW)], slab_v)

        lanes = lax.iota(jnp.int32, 16)

        def transpose_step(t, carry):
            s = t // (BPW // 16)
            j = t % (BPW // 16)
            vals = slab_v[s, pl.ds(j * 16, 16)]
            pos = (j * 16 + lanes) * NS + s
            plsc.store_scatter(idx_v, [pos >> 7, pos & 127], vals)
            return carry

        lax.fori_loop(0, NS * (BPW // 16), transpose_step, 0)

        for c in range(N_CHUNKS):
            buf = c % 2
            r0 = c * CHUNK
            if c >= 2:
                pltpu.make_async_copy(
                    rows_v.at[buf],
                    out_hbm.at[pl.ds(base_r + (c - 2) * CHUNK, CHUNK)],
                    sem_o,
                ).wait()

            def fire(j, carry, buf=buf, r0=r0):
                pltpu.async_copy(
                    table_hbm.at[idx_v.at[r0 + j]], rows_v.at[buf, j], sem_g)
                return carry

            lax.fori_loop(0, CHUNK, fire, 0)
            pltpu.make_async_copy(
                out_hbm.at[pl.ds(base_r + r0, CHUNK)], rows_v.at[buf], sem_g,
            ).wait()
            pltpu.async_copy(
                rows_v.at[buf], out_hbm.at[pl.ds(base_r + r0, CHUNK)], sem_o)

        for c in (N_CHUNKS - 2, N_CHUNKS - 1):
            pltpu.make_async_copy(
                rows_v.at[c % 2],
                out_hbm.at[pl.ds(base_r + c * CHUNK, CHUNK)],
                sem_o,
            ).wait()

    return k(table, sparse)


def _dotT(x, w):
    return lax.dot_general(x, w, (((1,), (1,)), ((), ())),
                           preferred_element_type=jnp.float32)


def _mlp_body(emb_ref, den_ref, w0e, w0d, b0r, w1, b1r, w2, b2r, w3, b3r,
              w4, ww, blast_ref, out_ref):
    x = emb_ref[...]
    d = den_ref[...]
    h = _dotT(x, w0e[...]) + _dotT(d, w0d[...]) + b0r[...]
    h = jnp.maximum(h, 0.0)
    h = jnp.maximum(_dotT(h, w1[...]) + b1r[...], 0.0)
    h = jnp.maximum(_dotT(h, w2[...]) + b2r[...], 0.0)
    h = jnp.maximum(_dotT(h, w3[...]) + b3r[...], 0.0)
    y = _dotT(h, w4[...])
    wide = _dotT(d, ww[...])
    out_ref[...] = y + wide + blast_ref[0]


def _tc_mlp(emb, dense, w0e, w0d, b0, w1, b1, w2, b2, w3, b3, w4, ww, blast):
    BM = 2048
    grid = (B // BM,)

    def const(shape):
        return pl.BlockSpec(shape, lambda i: (0, 0))

    return pl.pallas_call(
        _mlp_body,
        grid=grid,
        in_specs=[
            pl.BlockSpec((BM, NS * D), lambda i: (i, 0)),
            pl.BlockSpec((BM, DENSE), lambda i: (i, 0)),
            const((512, NS * D)),
            const((512, DENSE)),
            const((1, 512)),
            const((256, 512)),
            const((1, 256)),
            const((128, 256)),
            const((1, 128)),
            const((32, 128)),
            const((1, 32)),
            const((1, 32)),
            const((1, DENSE)),
            pl.BlockSpec(memory_space=pltpu.SMEM),
        ],
        out_specs=pl.BlockSpec((BM, 1), lambda i: (i, 0)),
        out_shape=jax.ShapeDtypeStruct((B, 1), jnp.float32),
    )(emb, dense, w0e, w0d, b0, w1, b1, w2, b2, w3, b3, w4, ww, blast)


def kernel(dense_feature, sparse_feature, table, Ww, bw, W0, b0, W1, b1, W2,
           b2, W3, b3, W4, b4):
    emb3d = _sc_gather(table, sparse_feature)
    emb = emb3d.reshape(B, NS * D)
    out = _tc_mlp(
        emb, dense_feature,
        W0[:, : NS * D], W0[:, NS * D:], b0.reshape(1, -1),
        W1, b1.reshape(1, -1),
        W2, b2.reshape(1, -1),
        W3, b3.reshape(1, -1),
        W4, Ww, (b4 + bw).reshape(1),
    )
    return out


# speedup vs baseline: 25.5244x; 1.6220x over previous
import functools

import jax
import jax.numpy as jnp
from jax import lax
from jax.experimental import pallas as pl
from jax.experimental.pallas import tpu as pltpu
from jax.experimental.pallas import tpu_sc as plsc

B = 16384
V = 1000000
D = 16
NS = 26
DENSE = 13

TCB = 8192
TGRID = 123
VPAD = TGRID * TCB

NW = 32
BPW = B // NW
IDX_COLS = 128
RPW = (BPW * NS) // IDX_COLS
CHUNK = 8
N_CHUNKS = RPW // CHUNK


def _tr_body(in_ref, out_ref):
    z = in_ref[...].T
    out_ref[...] = jnp.pad(z, ((0, 0), (0, 112)))


def _tc_repack_table(tT):
    return pl.pallas_call(
        _tr_body,
        grid=(TGRID,),
        in_specs=[pl.BlockSpec((16, TCB), lambda i: (0, i))],
        out_specs=pl.BlockSpec((TCB, 128), lambda i: (i, 0)),
        out_shape=jax.ShapeDtypeStruct((VPAD, 128), jnp.float32),
    )(tT)


def _sc_gather(table, sparse):
    mesh = plsc.VectorSubcoreMesh(core_axis_name="c", subcore_axis_name="s")
    out_rows = NW * RPW

    @functools.partial(
        pl.kernel,
        mesh=mesh,
        out_type=jax.ShapeDtypeStruct((out_rows, IDX_COLS, D), jnp.float32),
        scratch_types=[
            pltpu.VMEM((NS, BPW), jnp.int32),
            pltpu.VMEM((RPW, IDX_COLS), jnp.int32),
            pltpu.VMEM((2, CHUNK, IDX_COLS, D), jnp.float32),
            pltpu.SemaphoreType.DMA,
            pltpu.SemaphoreType.DMA,
        ],
        compiler_params=pltpu.CompilerParams(
            use_tc_tiling_on_sc=False, needs_layout_passes=False),
    )
    def k(table_hbm, idx_hbm, out_hbm, slab_v, idx_v, rows_v, sem_g, sem_o):
        wid = lax.axis_index("s") * 2 + lax.axis_index("c")
        base_b = wid * BPW
        base_r = wid * RPW

        pltpu.sync_copy(idx_hbm.at[:, pl.ds(base_b, BPW)], slab_v)

        lanes = lax.iota(jnp.int32, 16)

        def transpose_step(t, carry):
            s = t // (BPW // 16)
            j = t % (BPW // 16)
            vals = slab_v[s, pl.ds(j * 16, 16)] * 8
            pos = (j * 16 + lanes) * NS + s
            plsc.store_scatter(idx_v, [pos >> 7, pos & 127], vals)
            return carry

        lax.fori_loop(0, NS * (BPW // 16), transpose_step, 0)

        for c in range(N_CHUNKS):
            buf = c % 2
            r0 = c * CHUNK
            if c >= 2:
                pltpu.make_async_copy(
                    rows_v.at[buf],
                    out_hbm.at[pl.ds(base_r + (c - 2) * CHUNK, CHUNK)],
                    sem_o,
                ).wait()

            def fire(j, carry, buf=buf, r0=r0):
                pltpu.async_copy(
                    table_hbm.at[idx_v.at[r0 + j]], rows_v.at[buf, j], sem_g)
                return carry

            lax.fori_loop(0, CHUNK, fire, 0)
            pltpu.make_async_copy(
                out_hbm.at[pl.ds(base_r + r0, CHUNK)], rows_v.at[buf], sem_g,
            ).wait()
            pltpu.async_copy(
                rows_v.at[buf], out_hbm.at[pl.ds(base_r + r0, CHUNK)], sem_o)

        for c in (N_CHUNKS - 2, N_CHUNKS - 1):
            pltpu.make_async_copy(
                rows_v.at[c % 2],
                out_hbm.at[pl.ds(base_r + c * CHUNK, CHUNK)],
                sem_o,
            ).wait()

    return k(table, sparse)


def _dotT(x, w):
    return lax.dot_general(x, w, (((1,), (1,)), ((), ())),
                           preferred_element_type=jnp.float32)


def _mlp_body(emb_ref, den_ref, w0e, w0d, b0r, w1, b1r, w2, b2r, w3, b3r,
              w4, ww, blast_ref, out_ref):
    x = emb_ref[...]
    d = den_ref[...]
    h = _dotT(x, w0e[...]) + _dotT(d, w0d[...]) + b0r[...]
    h = jnp.maximum(h, 0.0)
    h = jnp.maximum(_dotT(h, w1[...]) + b1r[...], 0.0)
    h = jnp.maximum(_dotT(h, w2[...]) + b2r[...], 0.0)
    h = jnp.maximum(_dotT(h, w3[...]) + b3r[...], 0.0)
    y = _dotT(h, w4[...])
    wide = _dotT(d, ww[...])
    out_ref[...] = y + wide + blast_ref[0]


def _tc_mlp(emb, dense, w0e, w0d, b0, w1, b1, w2, b2, w3, b3, w4, ww, blast):
    BM = 2048
    grid = (B // BM,)

    def const(shape):
        return pl.BlockSpec(shape, lambda i: (0, 0))

    return pl.pallas_call(
        _mlp_body,
        grid=grid,
        in_specs=[
            pl.BlockSpec((BM, NS * D), lambda i: (i, 0)),
            pl.BlockSpec((BM, DENSE), lambda i: (i, 0)),
            const((512, NS * D)),
            const((512, DENSE)),
            const((1, 512)),
            const((256, 512)),
            const((1, 256)),
            const((128, 256)),
            const((1, 128)),
            const((32, 128)),
            const((1, 32)),
            const((1, 32)),
            const((1, DENSE)),
            pl.BlockSpec(memory_space=pltpu.SMEM),
        ],
        out_specs=pl.BlockSpec((BM, 1), lambda i: (i, 0)),
        out_shape=jax.ShapeDtypeStruct((B, 1), jnp.float32),
    )(emb, dense, w0e, w0d, b0, w1, b1, w2, b2, w3, b3, w4, ww, blast)


def kernel(dense_feature, sparse_feature, table, Ww, bw, W0, b0, W1, b1, W2,
           b2, W3, b3, W4, b4):
    table_rm = _tc_repack_table(table.T)
    table8 = table_rm.reshape(-1).reshape(VPAD * 8, D)
    emb3d = _sc_gather(table8, sparse_feature)
    emb = emb3d.reshape(B, NS * D)
    out = _tc_mlp(
        emb, dense_feature,
        W0[:, : NS * D], W0[:, NS * D:], b0.reshape(1, -1),
        W1, b1.reshape(1, -1),
        W2, b2.reshape(1, -1),
        W3, b3.reshape(1, -1),
        W4, Ww, (b4 + bw).reshape(1),
    )
    return out
